# Initial kernel scaffold; baseline (speedup 1.0000x reference)
import jax
import jax.numpy as jnp
from jax.experimental import pallas as pl


def kernel(risk_scores, events, survival_times):
    raise NotImplementedError("write your pallas kernel here")


# speedup vs baseline: 1.0233x; 1.0233x over previous
import functools

import jax
import jax.numpy as jnp
from jax import lax
from jax.experimental import pallas as pl
from jax.experimental.pallas import tpu as pltpu
from jax.experimental.pallas import tpu_sc as plsc

N = 1048576
_INFO = plsc.get_sparse_core_info()
NC = _INFO.num_cores
NS = _INFO.num_subcores
NW = NC * NS
CH = N // NW
G = 128
LN2 = 0.6931471805599453

_MESH = plsc.VectorSubcoreMesh(core_axis_name="c", subcore_axis_name="s")


_GATHER_DNUMS = lax.GatherDimensionNumbers(
    offset_dims=(), collapsed_slice_dims=(0,), start_index_map=(0,)
)


def _lane_bcast_last(x):
    idx = jnp.full((16, 1), 15, jnp.int32)
    return lax.gather(
        x, idx, _GATHER_DNUMS, slice_sizes=(1,),
        mode=lax.GatherScatterMode.PROMISE_IN_BOUNDS,
    )


_CPARAMS = pltpu.CompilerParams(needs_layout_passes=False)


@functools.partial(
    pl.kernel,
    mesh=_MESH,
    compiler_params=_CPARAMS,
    out_type=(
        jax.ShapeDtypeStruct((N,), jnp.float32),
        jax.ShapeDtypeStruct((NW, 4, 16), jnp.float32),
    ),
    scratch_types=[
        pltpu.VMEM((CH,), jnp.int32),
        pltpu.VMEM((CH,), jnp.int32),
        pltpu.VMEM((CH,), jnp.float32),
        pltpu.VMEM((4, 16), jnp.float32),
        pltpu.SemaphoreType.DMA,
    ],
)
def _phase1(idx_hbm, packed_hbm, w_hbm, part_hbm, idx_v, p_v, w_v, part_v, sem):
    wid = lax.axis_index("s") * NC + lax.axis_index("c")
    base = wid * CH
    pltpu.sync_copy(idx_hbm.at[pl.ds(base, CH)], idx_v)

    def gather_blk(j, c):
        pltpu.async_copy(
            packed_hbm.at[idx_v.at[pl.ds(j * G, G)]],
            p_v.at[pl.ds(j * G, G)],
            sem,
        ).wait()
        return c

    lax.fori_loop(0, CH // G, gather_blk, 0, unroll=False)

    def body(k, accs):
        aw, ae, aer = accs
        bits = p_v[pl.ds(k * 16, 16)]
        ef = (bits & 1).astype(jnp.float32)
        rr = plsc.bitcast(bits & -2, jnp.float32)
        ww = jnp.exp(rr)
        w_v[pl.ds(k * 16, 16)] = ww * (1.0 - 2.0 * ef)
        return (aw + ww, ae + ef, aer + ef * rr)

    z = jnp.zeros((16,), jnp.float32)
    aw, ae, aer = lax.fori_loop(0, CH // 16, body, (z, z, z), unroll=False)
    part_v[0, :] = aw
    part_v[1, :] = ae
    part_v[2, :] = aer
    part_v[3, :] = z
    pltpu.sync_copy(w_v, w_hbm.at[pl.ds(base, CH)])
    pltpu.sync_copy(part_v, part_hbm.at[wid])


@functools.partial(
    pl.kernel,
    mesh=_MESH,
    compiler_params=_CPARAMS,
    out_type=jax.ShapeDtypeStruct((NW, 16), jnp.float32),
    scratch_types=[
        pltpu.VMEM((CH,), jnp.float32),
        pltpu.VMEM((NW, 4, 16), jnp.float32),
        pltpu.VMEM((16,), jnp.float32),
        pltpu.SemaphoreType.DMA,
    ],
)
def _phase2(w_hbm, part_hbm, out_hbm, w_v, part_v, out_v, sem):
    wid = lax.axis_index("s") * NC + lax.axis_index("c")
    base = wid * CH
    pltpu.sync_copy(w_hbm.at[pl.ds(base, CH)], w_v)
    pltpu.sync_copy(part_hbm, part_v)

    wid_vec = jnp.full((16,), wid, jnp.int32)
    pacc = jnp.zeros((16,), jnp.float32)
    for v in range(NW):
        sel = jnp.full((16,), v, jnp.int32) < wid_vec
        pacc = pacc + jnp.where(sel, part_v[v, 0, :], 0.0)
    carry0 = _lane_bcast_last(jnp.cumsum(pacc))

    def body(k, st):
        cvec, acc = st
        swv = w_v[pl.ds(k * 16, 16)]
        b = plsc.bitcast(swv, jnp.int32)
        ww = plsc.bitcast(b & 0x7FFFFFFF, jnp.float32)
        ef = lax.shift_right_logical(b, 31).astype(jnp.float32)
        pre = jnp.cumsum(ww) + cvec
        cnew = _lane_bcast_last(pre)
        pb = plsc.bitcast(pre, jnp.int32)
        ex = lax.shift_right_logical(pb, 23) - 127
        m = plsc.bitcast((pb & 0x7FFFFF) | 0x3F800000, jnp.float32)
        big = m >= 1.5
        m = jnp.where(big, m * 0.5, m)
        exf = (ex + big.astype(jnp.int32)).astype(jnp.float32)
        s = (m - 1.0) / (m + 1.0)
        s2 = s * s
        lnm = 2.0 * s * (1.0 + s2 * (1.0 / 3.0 + s2 * 0.2))
        lnx = exf * LN2 + lnm
        return (cnew, acc + ef * lnx)

    _, acc = lax.fori_loop(
        0, CH // 16, body, (carry0, jnp.zeros((16,), jnp.float32)), unroll=False
    )
    out_v[...] = acc
    pltpu.sync_copy(out_v, out_hbm.at[wid])


def kernel(risk_scores, events, survival_times):
    idx = jnp.argsort(-survival_times).astype(jnp.int32)
    rbits = lax.bitcast_convert_type(risk_scores, jnp.int32)
    packed = (rbits & -2) | events
    w_signed, partials = _phase1(idx, packed)
    accs = _phase2(w_signed, partials)
    sum_e = partials[:, 1, :].sum()
    sum_er = partials[:, 2, :].sum()
    sum_elogc = accs.sum()
    return (sum_elogc - sum_er) / sum_e


# speedup vs baseline: 1.0233x; 1.0000x over previous
import functools

import jax
import jax.numpy as jnp
from jax import lax
from jax.experimental import pallas as pl
from jax.experimental.pallas import tpu as pltpu
from jax.experimental.pallas import tpu_sc as plsc

N = 1048576
_INFO = plsc.get_sparse_core_info()
NC = _INFO.num_cores
NS = _INFO.num_subcores
NW = NC * NS
CH = N // NW
G = 128
LN2 = 0.6931471805599453

_MESH = plsc.VectorSubcoreMesh(core_axis_name="c", subcore_axis_name="s")


_GATHER_DNUMS = lax.GatherDimensionNumbers(
    offset_dims=(), collapsed_slice_dims=(0,), start_index_map=(0,)
)


def _lane_bcast_last(x):
    idx = jnp.full((16, 1), 15, jnp.int32)
    return lax.gather(
        x, idx, _GATHER_DNUMS, slice_sizes=(1,),
        mode=lax.GatherScatterMode.PROMISE_IN_BOUNDS,
    )


_CPARAMS = pltpu.CompilerParams(needs_layout_passes=False)


@functools.partial(
    pl.kernel,
    mesh=_MESH,
    compiler_params=_CPARAMS,
    out_type=(
        jax.ShapeDtypeStruct((N,), jnp.float32),
        jax.ShapeDtypeStruct((NW, 4, 16), jnp.float32),
    ),
    scratch_types=[
        pltpu.VMEM((CH,), jnp.int32),
        pltpu.VMEM((CH,), jnp.int32),
        pltpu.VMEM((CH,), jnp.float32),
        pltpu.VMEM((4, 16), jnp.float32),
        pltpu.SemaphoreType.DMA,
    ],
)
def _phase1(idx_hbm, packed_hbm, w_hbm, part_hbm, idx_v, p_v, w_v, part_v, sem):
    wid = lax.axis_index("s") * NC + lax.axis_index("c")
    base = wid * CH
    pltpu.sync_copy(idx_hbm.at[pl.ds(base, CH)], idx_v)

    def gather_blk(j, c):
        pltpu.async_copy(
            packed_hbm.at[idx_v.at[pl.ds(j * G, G)]],
            p_v.at[pl.ds(j * G, G)],
            sem,
        ).wait()
        return c

    lax.fori_loop(0, CH // G, gather_blk, 0, unroll=False)

    def body(k, accs):
        aw, ae, aer = accs
        bits = p_v[pl.ds(k * 16, 16)]
        ef = (bits & 1).astype(jnp.float32)
        rr = plsc.bitcast(bits & -2, jnp.float32)
        ww = jnp.exp(rr)
        w_v[pl.ds(k * 16, 16)] = ww * (1.0 - 2.0 * ef)
        return (aw + ww, ae + ef, aer + ef * rr)

    z = jnp.zeros((16,), jnp.float32)
    aw, ae, aer = lax.fori_loop(0, CH // 16, body, (z, z, z), unroll=False)
    part_v[0, :] = aw
    part_v[1, :] = ae
    part_v[2, :] = aer
    part_v[3, :] = z
    pltpu.sync_copy(w_v, w_hbm.at[pl.ds(base, CH)])
    pltpu.sync_copy(part_v, part_hbm.at[wid])


@functools.partial(
    pl.kernel,
    mesh=_MESH,
    compiler_params=_CPARAMS,
    out_type=jax.ShapeDtypeStruct((NW, 16), jnp.float32),
    scratch_types=[
        pltpu.VMEM((CH,), jnp.float32),
        pltpu.VMEM((NW, 4, 16), jnp.float32),
        pltpu.VMEM((16,), jnp.float32),
        pltpu.SemaphoreType.DMA,
    ],
)
def _phase2(w_hbm, part_hbm, out_hbm, w_v, part_v, out_v, sem):
    wid = lax.axis_index("s") * NC + lax.axis_index("c")
    base = wid * CH
    pltpu.sync_copy(w_hbm.at[pl.ds(base, CH)], w_v)
    pltpu.sync_copy(part_hbm, part_v)

    wid_vec = jnp.full((16,), wid, jnp.int32)
    pacc = jnp.zeros((16,), jnp.float32)
    for v in range(NW):
        sel = jnp.full((16,), v, jnp.int32) < wid_vec
        pacc = pacc + jnp.where(sel, part_v[v, 0, :], 0.0)
    carry0 = _lane_bcast_last(jnp.cumsum(pacc))

    def body(k, st):
        cvec, acc = st
        swv = w_v[pl.ds(k * 16, 16)]
        b = plsc.bitcast(swv, jnp.int32)
        ww = plsc.bitcast(b & 0x7FFFFFFF, jnp.float32)
        ef = lax.shift_right_logical(b, 31).astype(jnp.float32)
        pre = jnp.cumsum(ww) + cvec
        cnew = _lane_bcast_last(pre)
        pb = plsc.bitcast(pre, jnp.int32)
        ex = lax.shift_right_logical(pb, 23) - 127
        m = plsc.bitcast((pb & 0x7FFFFF) | 0x3F800000, jnp.float32)
        big = m >= 1.5
        m = jnp.where(big, m * 0.5, m)
        exf = (ex + big.astype(jnp.int32)).astype(jnp.float32)
        s = (m - 1.0) / (m + 1.0)
        s2 = s * s
        lnm = 2.0 * s * (1.0 + s2 * (1.0 / 3.0 + s2 * 0.2))
        lnx = exf * LN2 + lnm
        return (cnew, acc + ef * lnx)

    _, acc = lax.fori_loop(
        0, CH // 16, body, (carry0, jnp.zeros((16,), jnp.float32)), unroll=False
    )
    out_v[...] = acc
    pltpu.sync_copy(out_v, out_hbm.at[wid])


from jax.experimental.compute_on import compute_on


@compute_on("tpu_sparsecore")
@jax.jit
def _sc_argsort(neg_t):
    return jnp.argsort(neg_t)


def kernel(risk_scores, events, survival_times):
    idx = _sc_argsort(-survival_times).astype(jnp.int32)
    rbits = lax.bitcast_convert_type(risk_scores, jnp.int32)
    packed = (rbits & -2) | events
    w_signed, partials = _phase1(idx, packed)
    accs = _phase2(w_signed, partials)
    sum_e = partials[:, 1, :].sum()
    sum_er = partials[:, 2, :].sum()
    sum_elogc = accs.sum()
    return (sum_elogc - sum_er) / sum_e
